# Initial kernel scaffold; baseline (speedup 1.0000x reference)
import jax
import jax.numpy as jnp
from jax.experimental import pallas as pl


def kernel(x, expert_indices, expert_weights, W_up, W_down):
    raise NotImplementedError("write your pallas kernel here")


# speedup vs baseline: 5.5710x; 5.5710x over previous
import functools

import jax
import jax.numpy as jnp
from jax.experimental import pallas as pl
from jax.experimental.pallas import tpu as pltpu

N_EXPERTS = 8
D_MODEL = 1024
EXPERT_FF = 512
TOP_K = 2
T_STEPS = 4
N_TOK = 2048
BETA = 0.9
V_TH = 1.0

BN = 256


def _ffn_block(x_ref, mask_ref, wup_ref, wdn_ref, out_ref, s_scratch):
    e = pl.program_id(1)

    xb = x_ref[...]
    xf = xb.reshape(T_STEPS * BN, D_MODEL)
    wu = wup_ref[0]
    h = jax.lax.dot_general(
        xf, wu, (((1,), (1,)), ((), ())),
        preferred_element_type=jnp.float32)

    v = jnp.zeros((BN, EXPERT_FF), jnp.float32)
    for t in range(T_STEPS):
        v = BETA * v + h[t * BN:(t + 1) * BN, :]
        s = (v >= V_TH).astype(jnp.float32)
        v = v - s * V_TH
        s_scratch[t * BN:(t + 1) * BN, :] = s

    wd = wdn_ref[0]
    o = jax.lax.dot_general(
        s_scratch[...], wd, (((1,), (1,)), ((), ())),
        preferred_element_type=jnp.float32)

    mw = mask_ref[0, 0, :]
    v2 = jnp.zeros((BN, D_MODEL), jnp.float32)
    for t in range(T_STEPS):
        v2 = BETA * v2 + o[t * BN:(t + 1) * BN, :]
        s2 = (v2 >= V_TH).astype(jnp.float32)
        v2 = v2 - s2 * V_TH
        contrib = s2 * mw[:, None]

        @pl.when(e == 0)
        def _():
            out_ref[t] = contrib

        @pl.when(e != 0)
        def _():
            out_ref[t] += contrib


def kernel(x, expert_indices, expert_weights, W_up, W_down):
    T, N, D = x.shape
    E = W_up.shape[0]
    onehot = jax.nn.one_hot(expert_indices, E, dtype=x.dtype)
    mask = jnp.einsum('nke,nk->en', onehot, expert_weights)
    mask = mask.reshape(E, 1, N)

    nb = N // BN
    grid = (nb, E)

    out = pl.pallas_call(
        _ffn_block,
        grid=grid,
        in_specs=[
            pl.BlockSpec((T, BN, D), lambda n, e: (0, n, 0)),
            pl.BlockSpec((1, 1, BN), lambda n, e: (e, 0, n)),
            pl.BlockSpec((1, EXPERT_FF, D), lambda n, e: (e, 0, 0)),
            pl.BlockSpec((1, D, EXPERT_FF), lambda n, e: (e, 0, 0)),
        ],
        out_specs=pl.BlockSpec((T, BN, D), lambda n, e: (0, n, 0)),
        out_shape=jax.ShapeDtypeStruct((T, N, D), x.dtype),
        scratch_shapes=[pltpu.VMEM((T_STEPS * BN, EXPERT_FF), jnp.float32)],
    )(x, mask, W_up, W_down)
    return out
